# baseline (speedup 1.0000x reference)
import functools

import jax
import jax.numpy as jnp
from jax import lax
from jax.experimental import pallas as pl
from jax.experimental.pallas import tpu as pltpu
from jax.experimental.pallas import tpu_sc as plsc

N = 10000
E = 320000
D = 128
DH = D // 2
NC = 2
NS = 16
LANE = 16
SEG = 640
EPS = E // NS
CH = 80
NCH = EPS // CH
IBLK = 25
NIB = NCH // IBLK
NBUF = 6
SCAT_LAG = 3
PSC = SEG // 64


def _sc_body(x_hbm, row_hbm, col_hbm,
             p_hbm, w_hbm, xp_hbm, darr_hbm,
             ib_r, ib_c, dloc, acc2, wseg, psb, pso, bufs, out_sh,
             gsems, ssems, irsems, icsems, ipsems, opsems):
    cid = lax.axis_index("c")
    sid = lax.axis_index("s")
    wid = cid * NS + sid
    seg0 = jnp.minimum(sid * SEG, N - SEG)
    ones = jnp.ones((LANE,), jnp.float32)
    zeros = jnp.zeros((LANE,), jnp.float32)

    def _zero_d(i, c):
        dloc[pl.ds(i * LANE, LANE)] = zeros
        return c
    lax.fori_loop(0, N // LANE, _zero_d, 0)

    cnt_d = [None] * NIB
    cnt_d[0] = pltpu.async_copy(
        col_hbm.at[sid, pl.ds(0, IBLK)], ib_c.at[0], icsems[0])
    for s in range(NIB):
        cnt_d[s].wait()
        if s + 1 < NIB:
            cnt_d[s + 1] = pltpu.async_copy(
                col_hbm.at[sid, pl.ds((s + 1) * IBLK, IBLK)],
                ib_c.at[(s + 1) % 2], icsems[(s + 1) % 2])
        slot = s % 2

        def _count(i, c):
            idx = ib_c[slot, i // (CH // LANE),
                       pl.ds((i % (CH // LANE)) * LANE, LANE)]
            plsc.addupdate_scatter(dloc, [idx], ones)
            return c
        lax.fori_loop(0, IBLK * CH // LANE, _count, 0)

    pltpu.sync_copy(dloc, darr_hbm.at[wid])
    plsc.subcore_barrier()

    pltpu.sync_copy(darr_hbm.at[pl.ds(cid * NS, NS), pl.ds(seg0, SEG)], acc2)

    def _wcalc(i, c):
        s = pl.ds(i * LANE, LANE)
        v = acc2[0, s]
        for t in range(1, NS):
            v = v + acc2[t, s]
        bits = plsc.bitcast(v, jnp.int32)
        y = plsc.bitcast(jnp.int32(0x5F3759DF) - (bits >> 1), jnp.float32)
        for _ in range(3):
            y = y * (1.5 - 0.5 * v * y * y)
        wseg[s] = jnp.where(v > 0.5, y, 0.0)
        return c
    lax.fori_loop(0, SEG // LANE, _wcalc, 0)

    @pl.when(cid == 0)
    def _():
        pltpu.sync_copy(wseg, w_hbm.at[pl.ds(seg0, SEG)])

    def _ps_compute(slot, cbase):
        for r in range(64):
            widx = jnp.zeros((LANE,), jnp.int32) + (cbase * 64 + r)
            wv = plsc.load_gather(wseg, [widx])
            for k in range(DH // LANE):
                src = pl.ds(cid * DH + k * LANE, LANE)
                dst = pl.ds(k * LANE, LANE)
                pso[slot, r, dst] = psb[slot, r, src] * wv

    pltpu.async_copy(x_hbm.at[pl.ds(seg0, 64)], psb.at[0], ipsems[0])

    def _ps_body(c, carry):
        pltpu.make_async_copy(x_hbm.at[pl.ds(seg0, 64)], psb.at[0],
                              ipsems[0]).wait()
        pltpu.async_copy(x_hbm.at[pl.ds(seg0 + (2 * c + 1) * 64, 64)],
                         psb.at[1], ipsems[1])

        @pl.when(c > 0)
        def _():
            pltpu.make_async_copy(pso.at[0], xp_hbm.at[pl.ds(0, 64)],
                                  opsems[0]).wait()
        _ps_compute(0, 2 * c)
        pltpu.async_copy(
            pso.at[0], xp_hbm.at[pl.ds(cid * N + seg0 + (2 * c) * 64, 64)],
            opsems[0])

        pltpu.make_async_copy(x_hbm.at[pl.ds(seg0, 64)], psb.at[1],
                              ipsems[1]).wait()

        @pl.when(c < PSC // 2 - 1)
        def _():
            pltpu.async_copy(x_hbm.at[pl.ds(seg0 + (2 * c + 2) * 64, 64)],
                             psb.at[0], ipsems[0])

        @pl.when(c > 0)
        def _():
            pltpu.make_async_copy(pso.at[1], xp_hbm.at[pl.ds(0, 64)],
                                  opsems[1]).wait()
        _ps_compute(1, 2 * c + 1)
        pltpu.async_copy(
            pso.at[1], xp_hbm.at[pl.ds(cid * N + seg0 + (2 * c + 1) * 64, 64)],
            opsems[1])
        return carry
    lax.fori_loop(0, PSC // 2, _ps_body, 0)
    pltpu.make_async_copy(pso.at[0], xp_hbm.at[pl.ds(0, 64)], opsems[0]).wait()
    pltpu.make_async_copy(pso.at[1], xp_hbm.at[pl.ds(0, 64)], opsems[1]).wait()

    def _zb(i, c):
        bufs[0][i // (DH // LANE), pl.ds((i % (DH // LANE)) * LANE, LANE)] = zeros
        return c
    lax.fori_loop(0, CH * DH // LANE, _zb, 0)
    for q in range(SEG // CH):
        pltpu.sync_copy(bufs[0], out_sh.at[pl.ds(seg0 + q * CH, CH)])
    plsc.subcore_barrier()

    def _adjust(slot):
        off = jnp.zeros((LANE,), jnp.int32) + cid * N

        def _adj(i, c):
            s = pl.ds((i % (CH // LANE)) * LANE, LANE)
            r = i // (CH // LANE)
            ib_r[slot, r, s] = ib_r[slot, r, s] + off
            return c
        lax.fori_loop(0, IBLK * CH // LANE, _adj, 0)

    ird = [None] * NIB
    icd = [None] * NIB
    for s in range(min(2, NIB)):
        ird[s] = pltpu.async_copy(
            row_hbm.at[sid, pl.ds(s * IBLK, IBLK)], ib_r.at[s % 3], irsems[s % 3])
        icd[s] = pltpu.async_copy(
            col_hbm.at[sid, pl.ds(s * IBLK, IBLK)], ib_c.at[s % 3], icsems[s % 3])
        ird[s].wait()
        icd[s].wait()
        _adjust(s % 3)

    gat = [None] * NCH
    scat = [None] * NCH
    for b in range(NBUF):
        gat[b] = pltpu.async_copy(xp_hbm.at[ib_r.at[0, b]], bufs[b], gsems[b])
    for j in range(NCH):
        s, jj = divmod(j, IBLK)
        if jj == 0 and 0 < s < NIB - 1:
            ird[s + 1].wait()
            icd[s + 1].wait()
            _adjust((s + 1) % 3)
        if jj == SCAT_LAG and s + 2 < NIB:
            ird[s + 2] = pltpu.async_copy(
                row_hbm.at[sid, pl.ds((s + 2) * IBLK, IBLK)],
                ib_r.at[(s + 2) % 3], irsems[(s + 2) % 3])
            icd[s + 2] = pltpu.async_copy(
                col_hbm.at[sid, pl.ds((s + 2) * IBLK, IBLK)],
                ib_c.at[(s + 2) % 3], icsems[(s + 2) % 3])
        gat[j].wait()
        scat[j] = pltpu.async_copy(
            bufs[j % NBUF], out_sh.at[ib_c.at[s % 3, jj]], ssems[j % NBUF],
            add=True)
        k = j - SCAT_LAG
        if k >= 0:
            scat[k].wait()
            nxt = k + NBUF
            if nxt < NCH:
                sn, jn = divmod(nxt, IBLK)
                gat[nxt] = pltpu.async_copy(
                    xp_hbm.at[ib_r.at[sn % 3, jn]], bufs[nxt % NBUF],
                    gsems[nxt % NBUF])
    for k in range(NCH - SCAT_LAG, NCH):
        scat[k].wait()

    plsc.subcore_barrier()

    pltpu.sync_copy(out_sh.at[pl.ds(seg0, SEG)],
                    p_hbm.at[pl.ds(cid * N + seg0, SEG)])


_sc_aggregate = functools.partial(
    pl.kernel,
    out_type=(
        jax.ShapeDtypeStruct((NC * N, DH), jnp.float32),
        jax.ShapeDtypeStruct((N,), jnp.float32),
        jax.ShapeDtypeStruct((NC * N, DH), jnp.float32),
        jax.ShapeDtypeStruct((NC * NS, N), jnp.float32),
    ),
    mesh=plsc.VectorSubcoreMesh(core_axis_name="c", subcore_axis_name="s",
                                num_cores=NC, num_subcores=NS),
    compiler_params=pltpu.CompilerParams(needs_layout_passes=False,
                                         use_tc_tiling_on_sc=False),
    scratch_types=[
        pltpu.VMEM((3, IBLK, CH), jnp.int32),
        pltpu.VMEM((3, IBLK, CH), jnp.int32),
        pltpu.VMEM((N,), jnp.float32),
        pltpu.VMEM((NS, SEG), jnp.float32),
        pltpu.VMEM((SEG,), jnp.float32),
        pltpu.VMEM((2, 64, D), jnp.float32),
        pltpu.VMEM((2, 64, DH), jnp.float32),
        [pltpu.VMEM((CH, DH), jnp.float32) for _ in range(NBUF)],
        pltpu.VMEM_SHARED((N, DH), jnp.float32),
        [pltpu.SemaphoreType.DMA for _ in range(NBUF)],
        [pltpu.SemaphoreType.DMA for _ in range(NBUF)],
        [pltpu.SemaphoreType.DMA for _ in range(3)],
        [pltpu.SemaphoreType.DMA for _ in range(3)],
        [pltpu.SemaphoreType.DMA for _ in range(2)],
        [pltpu.SemaphoreType.DMA for _ in range(2)],
    ],
)(_sc_body)


RB = 2000


def _tc_body(p0_ref, p1_ref, w_ref, wt_ref, o_ref):
    hi = jnp.concatenate([p0_ref[0], p1_ref[0]], axis=1) * w_ref[...]
    o_ref[...] = jnp.dot(hi, wt_ref[...], preferred_element_type=jnp.float32)


def _tc_combine(p3, w2, weight):
    return pl.pallas_call(
        _tc_body,
        grid=(N // RB,),
        in_specs=[
            pl.BlockSpec((1, RB, DH), lambda i: (0, i, 0)),
            pl.BlockSpec((1, RB, DH), lambda i: (1, i, 0)),
            pl.BlockSpec((RB, 1), lambda i: (i, 0)),
            pl.BlockSpec((D, D), lambda i: (0, 0)),
        ],
        out_specs=pl.BlockSpec((RB, D), lambda i: (i, 0)),
        out_shape=jax.ShapeDtypeStruct((N, D), jnp.float32),
    )(p3, p3, w2, weight)


@jax.jit
def kernel(x, adj, x0, weight):
    del x0
    row3 = adj[0].astype(jnp.int32).reshape(NS, NCH, CH)
    col3 = adj[1].astype(jnp.int32).reshape(NS, NCH, CH)
    p, w, _xp, _d = _sc_aggregate(x, row3, col3)
    return _tc_combine(p.reshape(NC, N, DH), w.reshape(N, 1), weight)


# speedup vs baseline: 1.1217x; 1.1217x over previous
import functools

import jax
import jax.numpy as jnp
from jax import lax
from jax.experimental import pallas as pl
from jax.experimental.pallas import tpu as pltpu
from jax.experimental.pallas import tpu_sc as plsc

N = 10000
E = 320000
D = 128
DH = D // 2
NC = 2
NS = 16
LANE = 16
SEG = 640
EPS = E // NS
CH = 80
NCH = EPS // CH
IBLK = 25
NIB = NCH // IBLK
NBUF = 6
SCAT_LAG = 1
PSC = SEG // 64


def _sc_body(x_hbm, row_hbm, col_hbm,
             p_hbm, w_hbm, xp_hbm, darr_hbm,
             ib_r, ib_c, dloc, acc2, wseg, psb, pso, bufs, out_sh,
             gsems, ssems, irsems, icsems, ipsems, opsems):
    cid = lax.axis_index("c")
    sid = lax.axis_index("s")
    wid = cid * NS + sid
    seg0 = jnp.minimum(sid * SEG, N - SEG)
    ones = jnp.ones((LANE,), jnp.float32)
    zeros = jnp.zeros((LANE,), jnp.float32)

    def _zero_d(i, c):
        dloc[pl.ds(i * LANE, LANE)] = zeros
        return c
    lax.fori_loop(0, N // LANE, _zero_d, 0)

    cnt_d = [None] * NIB
    cnt_d[0] = pltpu.async_copy(
        col_hbm.at[sid, pl.ds(0, IBLK)], ib_c.at[0], icsems[0])
    for s in range(NIB):
        cnt_d[s].wait()
        if s + 1 < NIB:
            cnt_d[s + 1] = pltpu.async_copy(
                col_hbm.at[sid, pl.ds((s + 1) * IBLK, IBLK)],
                ib_c.at[(s + 1) % 2], icsems[(s + 1) % 2])
        slot = s % 2

        def _count(i, c):
            idx = ib_c[slot, i // (CH // LANE),
                       pl.ds((i % (CH // LANE)) * LANE, LANE)]
            plsc.addupdate_scatter(dloc, [idx], ones)
            return c
        lax.fori_loop(0, IBLK * CH // LANE, _count, 0)

    pltpu.sync_copy(dloc, darr_hbm.at[wid])
    plsc.subcore_barrier()

    pltpu.sync_copy(darr_hbm.at[pl.ds(cid * NS, NS), pl.ds(seg0, SEG)], acc2)

    def _wcalc(i, c):
        s = pl.ds(i * LANE, LANE)
        v = acc2[0, s]
        for t in range(1, NS):
            v = v + acc2[t, s]
        bits = plsc.bitcast(v, jnp.int32)
        y = plsc.bitcast(jnp.int32(0x5F3759DF) - (bits >> 1), jnp.float32)
        for _ in range(3):
            y = y * (1.5 - 0.5 * v * y * y)
        wseg[s] = jnp.where(v > 0.5, y, 0.0)
        return c
    lax.fori_loop(0, SEG // LANE, _wcalc, 0)

    @pl.when(cid == 0)
    def _():
        pltpu.sync_copy(wseg, w_hbm.at[pl.ds(seg0, SEG)])

    def _ps_compute(slot, cbase):
        for r in range(64):
            widx = jnp.zeros((LANE,), jnp.int32) + (cbase * 64 + r)
            wv = plsc.load_gather(wseg, [widx])
            for k in range(DH // LANE):
                src = pl.ds(cid * DH + k * LANE, LANE)
                dst = pl.ds(k * LANE, LANE)
                pso[slot, r, dst] = psb[slot, r, src] * wv

    pltpu.async_copy(x_hbm.at[pl.ds(seg0, 64)], psb.at[0], ipsems[0])

    def _ps_body(c, carry):
        pltpu.make_async_copy(x_hbm.at[pl.ds(seg0, 64)], psb.at[0],
                              ipsems[0]).wait()
        pltpu.async_copy(x_hbm.at[pl.ds(seg0 + (2 * c + 1) * 64, 64)],
                         psb.at[1], ipsems[1])

        @pl.when(c > 0)
        def _():
            pltpu.make_async_copy(pso.at[0], xp_hbm.at[pl.ds(0, 64)],
                                  opsems[0]).wait()
        _ps_compute(0, 2 * c)
        pltpu.async_copy(
            pso.at[0], xp_hbm.at[pl.ds(cid * N + seg0 + (2 * c) * 64, 64)],
            opsems[0])

        pltpu.make_async_copy(x_hbm.at[pl.ds(seg0, 64)], psb.at[1],
                              ipsems[1]).wait()

        @pl.when(c < PSC // 2 - 1)
        def _():
            pltpu.async_copy(x_hbm.at[pl.ds(seg0 + (2 * c + 2) * 64, 64)],
                             psb.at[0], ipsems[0])

        @pl.when(c > 0)
        def _():
            pltpu.make_async_copy(pso.at[1], xp_hbm.at[pl.ds(0, 64)],
                                  opsems[1]).wait()
        _ps_compute(1, 2 * c + 1)
        pltpu.async_copy(
            pso.at[1], xp_hbm.at[pl.ds(cid * N + seg0 + (2 * c + 1) * 64, 64)],
            opsems[1])
        return carry
    lax.fori_loop(0, PSC // 2, _ps_body, 0)
    pltpu.make_async_copy(pso.at[0], xp_hbm.at[pl.ds(0, 64)], opsems[0]).wait()
    pltpu.make_async_copy(pso.at[1], xp_hbm.at[pl.ds(0, 64)], opsems[1]).wait()

    def _zb(i, c):
        bufs[0][i // (DH // LANE), pl.ds((i % (DH // LANE)) * LANE, LANE)] = zeros
        return c
    lax.fori_loop(0, CH * DH // LANE, _zb, 0)
    for q in range(SEG // CH):
        pltpu.sync_copy(bufs[0], out_sh.at[pl.ds(seg0 + q * CH, CH)])
    plsc.subcore_barrier()

    def _adjust(slot):
        off = jnp.zeros((LANE,), jnp.int32) + cid * N

        def _adj(i, c):
            s = pl.ds((i % (CH // LANE)) * LANE, LANE)
            r = i // (CH // LANE)
            ib_r[slot, r, s] = ib_r[slot, r, s] + off
            return c
        lax.fori_loop(0, IBLK * CH // LANE, _adj, 0)

    ird = [None] * NIB
    icd = [None] * NIB
    for s in range(min(2, NIB)):
        ird[s] = pltpu.async_copy(
            row_hbm.at[sid, pl.ds(s * IBLK, IBLK)], ib_r.at[s % 3], irsems[s % 3])
        icd[s] = pltpu.async_copy(
            col_hbm.at[sid, pl.ds(s * IBLK, IBLK)], ib_c.at[s % 3], icsems[s % 3])
        ird[s].wait()
        icd[s].wait()
        _adjust(s % 3)

    gat = [None] * NCH
    scat = [None] * NCH
    for b in range(NBUF):
        gat[b] = pltpu.async_copy(xp_hbm.at[ib_r.at[0, b]], bufs[b], gsems[b])
    for j in range(NCH):
        s, jj = divmod(j, IBLK)
        if jj == 0 and 0 < s < NIB - 1:
            ird[s + 1].wait()
            icd[s + 1].wait()
            _adjust((s + 1) % 3)
        if jj == SCAT_LAG and s + 2 < NIB:
            ird[s + 2] = pltpu.async_copy(
                row_hbm.at[sid, pl.ds((s + 2) * IBLK, IBLK)],
                ib_r.at[(s + 2) % 3], irsems[(s + 2) % 3])
            icd[s + 2] = pltpu.async_copy(
                col_hbm.at[sid, pl.ds((s + 2) * IBLK, IBLK)],
                ib_c.at[(s + 2) % 3], icsems[(s + 2) % 3])
        gat[j].wait()
        scat[j] = pltpu.async_copy(
            bufs[j % NBUF], out_sh.at[ib_c.at[s % 3, jj]], ssems[j % NBUF],
            add=True)
        k = j - SCAT_LAG
        if k >= 0:
            scat[k].wait()
            nxt = k + NBUF
            if nxt < NCH:
                sn, jn = divmod(nxt, IBLK)
                gat[nxt] = pltpu.async_copy(
                    xp_hbm.at[ib_r.at[sn % 3, jn]], bufs[nxt % NBUF],
                    gsems[nxt % NBUF])
    for k in range(NCH - SCAT_LAG, NCH):
        scat[k].wait()

    plsc.subcore_barrier()

    pltpu.sync_copy(out_sh.at[pl.ds(seg0, SEG)],
                    p_hbm.at[pl.ds(cid * N + seg0, SEG)])


_sc_aggregate = functools.partial(
    pl.kernel,
    out_type=(
        jax.ShapeDtypeStruct((NC * N, DH), jnp.float32),
        jax.ShapeDtypeStruct((N,), jnp.float32),
        jax.ShapeDtypeStruct((NC * N, DH), jnp.float32),
        jax.ShapeDtypeStruct((NC * NS, N), jnp.float32),
    ),
    mesh=plsc.VectorSubcoreMesh(core_axis_name="c", subcore_axis_name="s",
                                num_cores=NC, num_subcores=NS),
    compiler_params=pltpu.CompilerParams(needs_layout_passes=False,
                                         use_tc_tiling_on_sc=False),
    scratch_types=[
        pltpu.VMEM((3, IBLK, CH), jnp.int32),
        pltpu.VMEM((3, IBLK, CH), jnp.int32),
        pltpu.VMEM((N,), jnp.float32),
        pltpu.VMEM((NS, SEG), jnp.float32),
        pltpu.VMEM((SEG,), jnp.float32),
        pltpu.VMEM((2, 64, D), jnp.float32),
        pltpu.VMEM((2, 64, DH), jnp.float32),
        [pltpu.VMEM((CH, DH), jnp.float32) for _ in range(NBUF)],
        pltpu.VMEM_SHARED((N, DH), jnp.float32),
        [pltpu.SemaphoreType.DMA for _ in range(NBUF)],
        [pltpu.SemaphoreType.DMA for _ in range(NBUF)],
        [pltpu.SemaphoreType.DMA for _ in range(3)],
        [pltpu.SemaphoreType.DMA for _ in range(3)],
        [pltpu.SemaphoreType.DMA for _ in range(2)],
        [pltpu.SemaphoreType.DMA for _ in range(2)],
    ],
)(_sc_body)


RB = 2000


def _tc_body(p0_ref, p1_ref, w_ref, wt_ref, o_ref):
    hi = jnp.concatenate([p0_ref[0], p1_ref[0]], axis=1) * w_ref[...]
    o_ref[...] = jnp.dot(hi, wt_ref[...], preferred_element_type=jnp.float32)


def _tc_combine(p3, w2, weight):
    return pl.pallas_call(
        _tc_body,
        grid=(N // RB,),
        in_specs=[
            pl.BlockSpec((1, RB, DH), lambda i: (0, i, 0)),
            pl.BlockSpec((1, RB, DH), lambda i: (1, i, 0)),
            pl.BlockSpec((RB, 1), lambda i: (i, 0)),
            pl.BlockSpec((D, D), lambda i: (0, 0)),
        ],
        out_specs=pl.BlockSpec((RB, D), lambda i: (i, 0)),
        out_shape=jax.ShapeDtypeStruct((N, D), jnp.float32),
    )(p3, p3, w2, weight)


@jax.jit
def kernel(x, adj, x0, weight):
    del x0
    row3 = adj[0].astype(jnp.int32).reshape(NS, NCH, CH)
    col3 = adj[1].astype(jnp.int32).reshape(NS, NCH, CH)
    p, w, _xp, _d = _sc_aggregate(x, row3, col3)
    return _tc_combine(p.reshape(NC, N, DH), w.reshape(N, 1), weight)


# speedup vs baseline: 1.1366x; 1.0133x over previous
import functools

import jax
import jax.numpy as jnp
from jax import lax
from jax.experimental import pallas as pl
from jax.experimental.pallas import tpu as pltpu
from jax.experimental.pallas import tpu_sc as plsc

N = 10000
E = 320000
D = 128
DH = D // 2
NC = 2
NS = 16
LANE = 16
SEG = 640
EPS = E // NS
CH = 80
NCH = EPS // CH
IBLK = 25
NIB = NCH // IBLK
NBUF = 6
SCAT_LAG = 1
PSC = SEG // 64


def _sc_body(x_hbm, row_hbm, col_hbm,
             p_hbm, w_hbm, xp_hbm, darr_hbm,
             ib_r, ib_c, dloc, acc2, wseg, psb, pso, bufs, out_sh,
             gsems, ssems, irsems, icsems, ipsems, opsems):
    cid = lax.axis_index("c")
    sid = lax.axis_index("s")
    wid = cid * NS + sid
    seg0 = jnp.minimum(sid * SEG, N - SEG)
    ones = jnp.ones((LANE,), jnp.float32)
    zeros = jnp.zeros((LANE,), jnp.float32)

    def _zero_d(i, c):
        dloc[pl.ds(i * LANE, LANE)] = zeros
        return c
    lax.fori_loop(0, N // LANE, _zero_d, 0)

    cnt_d = [None] * NIB
    cnt_d[0] = pltpu.async_copy(
        col_hbm.at[sid, pl.ds(0, IBLK)], ib_c.at[0], icsems[0])
    for s in range(NIB):
        cnt_d[s].wait()
        if s + 1 < NIB:
            cnt_d[s + 1] = pltpu.async_copy(
                col_hbm.at[sid, pl.ds((s + 1) * IBLK, IBLK)],
                ib_c.at[(s + 1) % 2], icsems[(s + 1) % 2])
        slot = s % 2

        def _count(i, c):
            idx = ib_c[slot, i // (CH // LANE),
                       pl.ds((i % (CH // LANE)) * LANE, LANE)]
            plsc.addupdate_scatter(dloc, [idx], ones)
            return c
        lax.fori_loop(0, IBLK * CH // LANE, _count, 0)

    pltpu.sync_copy(dloc, darr_hbm.at[wid])
    plsc.subcore_barrier()

    pltpu.sync_copy(darr_hbm.at[pl.ds(cid * NS, NS), pl.ds(seg0, SEG)], acc2)

    def _wcalc(i, c):
        s = pl.ds(i * LANE, LANE)
        v = acc2[0, s]
        for t in range(1, NS):
            v = v + acc2[t, s]
        bits = plsc.bitcast(v, jnp.int32)
        y = plsc.bitcast(jnp.int32(0x5F3759DF) - (bits >> 1), jnp.float32)
        for _ in range(3):
            y = y * (1.5 - 0.5 * v * y * y)
        wseg[s] = jnp.where(v > 0.5, y, 0.0)
        return c
    lax.fori_loop(0, SEG // LANE, _wcalc, 0)

    @pl.when(cid == 0)
    def _():
        pltpu.sync_copy(wseg, w_hbm.at[pl.ds(seg0, SEG)])

    def _zb(i, c):
        bufs[0][i // (DH // LANE), pl.ds((i % (DH // LANE)) * LANE, LANE)] = zeros
        return c
    lax.fori_loop(0, CH * DH // LANE, _zb, 0)
    zd = [pltpu.async_copy(bufs[0], out_sh.at[pl.ds(seg0 + q * CH, CH)],
                           ssems[q % NBUF]) for q in range(SEG // CH)]
    ird = [None] * NIB
    icd = [None] * NIB
    for s in range(min(2, NIB)):
        ird[s] = pltpu.async_copy(
            row_hbm.at[sid, pl.ds(s * IBLK, IBLK)], ib_r.at[s % 3], irsems[s % 3])
        icd[s] = pltpu.async_copy(
            col_hbm.at[sid, pl.ds(s * IBLK, IBLK)], ib_c.at[s % 3], icsems[s % 3])

    def _ps_compute(slot, cbase):
        for r in range(64):
            widx = jnp.zeros((LANE,), jnp.int32) + (cbase * 64 + r)
            wv = plsc.load_gather(wseg, [widx])
            for k in range(DH // LANE):
                src = pl.ds(cid * DH + k * LANE, LANE)
                dst = pl.ds(k * LANE, LANE)
                pso[slot, r, dst] = psb[slot, r, src] * wv

    pltpu.async_copy(x_hbm.at[pl.ds(seg0, 64)], psb.at[0], ipsems[0])

    def _ps_body(c, carry):
        pltpu.make_async_copy(x_hbm.at[pl.ds(seg0, 64)], psb.at[0],
                              ipsems[0]).wait()
        pltpu.async_copy(x_hbm.at[pl.ds(seg0 + (2 * c + 1) * 64, 64)],
                         psb.at[1], ipsems[1])

        @pl.when(c > 0)
        def _():
            pltpu.make_async_copy(pso.at[0], xp_hbm.at[pl.ds(0, 64)],
                                  opsems[0]).wait()
        _ps_compute(0, 2 * c)
        pltpu.async_copy(
            pso.at[0], xp_hbm.at[pl.ds(cid * N + seg0 + (2 * c) * 64, 64)],
            opsems[0])

        pltpu.make_async_copy(x_hbm.at[pl.ds(seg0, 64)], psb.at[1],
                              ipsems[1]).wait()

        @pl.when(c < PSC // 2 - 1)
        def _():
            pltpu.async_copy(x_hbm.at[pl.ds(seg0 + (2 * c + 2) * 64, 64)],
                             psb.at[0], ipsems[0])

        @pl.when(c > 0)
        def _():
            pltpu.make_async_copy(pso.at[1], xp_hbm.at[pl.ds(0, 64)],
                                  opsems[1]).wait()
        _ps_compute(1, 2 * c + 1)
        pltpu.async_copy(
            pso.at[1], xp_hbm.at[pl.ds(cid * N + seg0 + (2 * c + 1) * 64, 64)],
            opsems[1])
        return carry
    lax.fori_loop(0, PSC // 2, _ps_body, 0)
    pltpu.make_async_copy(pso.at[0], xp_hbm.at[pl.ds(0, 64)], opsems[0]).wait()
    pltpu.make_async_copy(pso.at[1], xp_hbm.at[pl.ds(0, 64)], opsems[1]).wait()

    for d in zd:
        d.wait()

    def _adjust(slot):
        off = jnp.zeros((LANE,), jnp.int32) + cid * N

        def _adj(i, c):
            s = pl.ds((i % (CH // LANE)) * LANE, LANE)
            r = i // (CH // LANE)
            ib_r[slot, r, s] = ib_r[slot, r, s] + off
            return c
        lax.fori_loop(0, IBLK * CH // LANE, _adj, 0)

    for s in range(min(2, NIB)):
        ird[s].wait()
        icd[s].wait()
        _adjust(s % 3)
    plsc.subcore_barrier()

    gat = [None] * NCH
    scat = [None] * NCH
    for b in range(NBUF):
        gat[b] = pltpu.async_copy(xp_hbm.at[ib_r.at[0, b]], bufs[b], gsems[b])
    for j in range(NCH):
        s, jj = divmod(j, IBLK)
        if jj == 0 and 0 < s < NIB - 1:
            ird[s + 1].wait()
            icd[s + 1].wait()
            _adjust((s + 1) % 3)
        if jj == SCAT_LAG and s + 2 < NIB:
            ird[s + 2] = pltpu.async_copy(
                row_hbm.at[sid, pl.ds((s + 2) * IBLK, IBLK)],
                ib_r.at[(s + 2) % 3], irsems[(s + 2) % 3])
            icd[s + 2] = pltpu.async_copy(
                col_hbm.at[sid, pl.ds((s + 2) * IBLK, IBLK)],
                ib_c.at[(s + 2) % 3], icsems[(s + 2) % 3])
        gat[j].wait()
        scat[j] = pltpu.async_copy(
            bufs[j % NBUF], out_sh.at[ib_c.at[s % 3, jj]], ssems[j % NBUF],
            add=True)
        k = j - SCAT_LAG
        if k >= 0:
            scat[k].wait()
            nxt = k + NBUF
            if nxt < NCH:
                sn, jn = divmod(nxt, IBLK)
                gat[nxt] = pltpu.async_copy(
                    xp_hbm.at[ib_r.at[sn % 3, jn]], bufs[nxt % NBUF],
                    gsems[nxt % NBUF])
    for k in range(NCH - SCAT_LAG, NCH):
        scat[k].wait()

    plsc.subcore_barrier()

    pltpu.sync_copy(out_sh.at[pl.ds(seg0, SEG)],
                    p_hbm.at[pl.ds(cid * N + seg0, SEG)])


_sc_aggregate = functools.partial(
    pl.kernel,
    out_type=(
        jax.ShapeDtypeStruct((NC * N, DH), jnp.float32),
        jax.ShapeDtypeStruct((N,), jnp.float32),
        jax.ShapeDtypeStruct((NC * N, DH), jnp.float32),
        jax.ShapeDtypeStruct((NC * NS, N), jnp.float32),
    ),
    mesh=plsc.VectorSubcoreMesh(core_axis_name="c", subcore_axis_name="s",
                                num_cores=NC, num_subcores=NS),
    compiler_params=pltpu.CompilerParams(needs_layout_passes=False,
                                         use_tc_tiling_on_sc=False),
    scratch_types=[
        pltpu.VMEM((3, IBLK, CH), jnp.int32),
        pltpu.VMEM((3, IBLK, CH), jnp.int32),
        pltpu.VMEM((N,), jnp.float32),
        pltpu.VMEM((NS, SEG), jnp.float32),
        pltpu.VMEM((SEG,), jnp.float32),
        pltpu.VMEM((2, 64, D), jnp.float32),
        pltpu.VMEM((2, 64, DH), jnp.float32),
        [pltpu.VMEM((CH, DH), jnp.float32) for _ in range(NBUF)],
        pltpu.VMEM_SHARED((N, DH), jnp.float32),
        [pltpu.SemaphoreType.DMA for _ in range(NBUF)],
        [pltpu.SemaphoreType.DMA for _ in range(NBUF)],
        [pltpu.SemaphoreType.DMA for _ in range(3)],
        [pltpu.SemaphoreType.DMA for _ in range(3)],
        [pltpu.SemaphoreType.DMA for _ in range(2)],
        [pltpu.SemaphoreType.DMA for _ in range(2)],
    ],
)(_sc_body)


RB = 2000


def _tc_body(p0_ref, p1_ref, w_ref, wt_ref, o_ref):
    hi = jnp.concatenate([p0_ref[0], p1_ref[0]], axis=1) * w_ref[...]
    o_ref[...] = jnp.dot(hi, wt_ref[...], preferred_element_type=jnp.float32)


def _tc_combine(p3, w2, weight):
    return pl.pallas_call(
        _tc_body,
        grid=(N // RB,),
        in_specs=[
            pl.BlockSpec((1, RB, DH), lambda i: (0, i, 0)),
            pl.BlockSpec((1, RB, DH), lambda i: (1, i, 0)),
            pl.BlockSpec((RB, 1), lambda i: (i, 0)),
            pl.BlockSpec((D, D), lambda i: (0, 0)),
        ],
        out_specs=pl.BlockSpec((RB, D), lambda i: (i, 0)),
        out_shape=jax.ShapeDtypeStruct((N, D), jnp.float32),
    )(p3, p3, w2, weight)


@jax.jit
def kernel(x, adj, x0, weight):
    del x0
    row3 = adj[0].astype(jnp.int32).reshape(NS, NCH, CH)
    col3 = adj[1].astype(jnp.int32).reshape(NS, NCH, CH)
    p, w, _xp, _d = _sc_aggregate(x, row3, col3)
    return _tc_combine(p.reshape(NC, N, DH), w.reshape(N, 1), weight)
